# Initial kernel scaffold; baseline (speedup 1.0000x reference)
import jax
import jax.numpy as jnp
from jax.experimental import pallas as pl


def kernel(hidden_states, router_logits, W13, W2, S13, S2, P13, P2):
    raise NotImplementedError("write your pallas kernel here")


# speedup vs baseline: 2.4207x; 2.4207x over previous
import functools

import jax
import jax.numpy as jnp
from jax.experimental import pallas as pl
from jax.experimental.pallas import tpu as pltpu

_E = 8
_H = 1024
_I = 2048
_G = 128
_T = 256

_BI = 1024
_J = _I // _BI
_NG = _H // _G
_NG2 = _BI // _G


def _moe_body(hs_ref, logits_ref, w1_ref, w3_ref, s1_ref, s3_ref, w2_ref,
              s2_ref, p13_ref, p2_ref, out_ref, w1s, w3s, w2s):
    e = pl.program_id(0)
    j = pl.program_id(1)

    @pl.when(jnp.logical_and(e == 0, j == 0))
    def _init():
        out_ref[...] = jnp.zeros_like(out_ref)

    logits = logits_ref[...]
    iota = jax.lax.broadcasted_iota(jnp.int32, logits.shape, 1)
    m1 = jnp.max(logits, axis=1, keepdims=True)
    a1 = jnp.min(jnp.where(logits == m1, iota, _E), axis=1, keepdims=True)
    masked = jnp.where(iota == a1, -jnp.inf, logits)
    m2 = jnp.max(masked, axis=1, keepdims=True)
    a2 = jnp.min(jnp.where(masked == m2, iota, _E), axis=1, keepdims=True)
    p_top = jax.nn.sigmoid(m1 - m2)
    p_snd = jax.nn.sigmoid(m2 - m1)
    fscale = (jnp.where(a1 == e, p_top, 0.0)
              + jnp.where(a2 == e, p_snd, 0.0))

    p13 = p13_ref[0, 0, 0]
    p2 = p2_ref[0, 0, 0]

    for g in range(_NG):
        sl = slice(g * _G, (g + 1) * _G)
        w1s[sl, :] = (w1_ref[0, sl, :].astype(jnp.float32)
                      * s1_ref[0, g, :][None, :]).astype(jnp.bfloat16)
        w3s[sl, :] = (w3_ref[0, sl, :].astype(jnp.float32)
                      * s3_ref[0, g, :][None, :]).astype(jnp.bfloat16)
    for g in range(_NG2):
        sl = slice(g * _G, (g + 1) * _G)
        w2s[sl, :] = (w2_ref[0, sl, :].astype(jnp.float32)
                      * s2_ref[0, 0, g, :][None, :]).astype(jnp.bfloat16)

    dot = functools.partial(
        jax.lax.dot_general,
        dimension_numbers=(((1,), (0,)), ((), ())),
        preferred_element_type=jnp.float32,
    )

    aq = (jnp.clip(hs_ref[...] / p13, -448.0, 448.0)
          .astype(jnp.float8_e4m3fn).astype(jnp.bfloat16))
    fc1 = dot(aq, w1s[...]) * p13
    gate = dot(aq, w3s[...]) * p13
    h2 = fc1 * (gate * jax.nn.sigmoid(gate))

    aq2 = (jnp.clip(h2 / p2, -448.0, 448.0)
           .astype(jnp.float8_e4m3fn).astype(jnp.bfloat16))
    fc2 = dot(aq2, w2s[...]) * p2

    out_ref[...] += fc2 * fscale


def kernel(hidden_states, router_logits, W13, W2, S13, S2, P13, P2):
    hs = hidden_states.reshape(-1, _H)
    S2r = S2.reshape(_E, _J, _NG2, _H)
    p13b = jnp.broadcast_to(P13[:, None, None], (_E, 8, 128))
    p2b = jnp.broadcast_to(P2[:, None, None], (_E, 8, 128))

    out = pl.pallas_call(
        _moe_body,
        grid=(_E, _J),
        in_specs=[
            pl.BlockSpec((_T, _H), lambda e, j: (0, 0)),
            pl.BlockSpec((_T, _E), lambda e, j: (0, 0)),
            pl.BlockSpec((1, _H, _BI), lambda e, j: (e, 0, j)),
            pl.BlockSpec((1, _H, _BI), lambda e, j: (e, 0, j + _J)),
            pl.BlockSpec((1, _NG, _BI), lambda e, j: (e, 0, j)),
            pl.BlockSpec((1, _NG, _BI), lambda e, j: (e, 0, j + _J)),
            pl.BlockSpec((1, _BI, _H), lambda e, j: (e, j, 0)),
            pl.BlockSpec((1, 1, _NG2, _H), lambda e, j: (e, j, 0, 0)),
            pl.BlockSpec((1, 8, 128), lambda e, j: (e, 0, 0)),
            pl.BlockSpec((1, 8, 128), lambda e, j: (e, 0, 0)),
        ],
        out_specs=pl.BlockSpec((_T, _H), lambda e, j: (0, 0)),
        out_shape=jax.ShapeDtypeStruct((_T, _H), jnp.float32),
        scratch_shapes=[
            pltpu.VMEM((_H, _BI), jnp.bfloat16),
            pltpu.VMEM((_H, _BI), jnp.bfloat16),
            pltpu.VMEM((_BI, _H), jnp.bfloat16),
        ],
        compiler_params=pltpu.CompilerParams(
            dimension_semantics=("arbitrary", "arbitrary")),
    )(hs, router_logits, W13, W13, S13, S13, W2, S2r, p13b, p2b)

    return out.reshape(hidden_states.shape)


# speedup vs baseline: 2.6461x; 1.0931x over previous
import functools

import jax
import jax.numpy as jnp
from jax.experimental import pallas as pl
from jax.experimental.pallas import tpu as pltpu

_E = 8
_H = 1024
_I = 2048
_G = 128
_T = 256

_NG = _H // _G
_NG2 = _I // _G


def _moe_body(hs_ref, logits_ref, w13_ref, s13_ref, w2_ref, s2_ref,
              p13_ref, p2_ref, out_ref, w1s, w3s, w2s):
    e = pl.program_id(0)

    @pl.when(e == 0)
    def _init():
        out_ref[...] = jnp.zeros_like(out_ref)

    logits = logits_ref[...]
    iota = jax.lax.broadcasted_iota(jnp.int32, logits.shape, 1)
    m1 = jnp.max(logits, axis=1, keepdims=True)
    a1 = jnp.min(jnp.where(logits == m1, iota, _E), axis=1, keepdims=True)
    masked = jnp.where(iota == a1, -jnp.inf, logits)
    m2 = jnp.max(masked, axis=1, keepdims=True)
    a2 = jnp.min(jnp.where(masked == m2, iota, _E), axis=1, keepdims=True)
    p_top = jax.nn.sigmoid(m1 - m2)
    p_snd = jax.nn.sigmoid(m2 - m1)
    fscale = (jnp.where(a1 == e, p_top, 0.0)
              + jnp.where(a2 == e, p_snd, 0.0))

    p13 = p13_ref[0, 0, 0]
    p2 = p2_ref[0, 0, 0]

    for g in range(_NG):
        sl = slice(g * _G, (g + 1) * _G)
        w1s[sl, :] = (w13_ref[0, sl, :_I].astype(jnp.float32)
                      * s13_ref[0, g, :_I][None, :]).astype(jnp.bfloat16)
        w3s[sl, :] = (w13_ref[0, sl, _I:].astype(jnp.float32)
                      * s13_ref[0, g, _I:][None, :]).astype(jnp.bfloat16)
    for g in range(_NG2):
        sl = slice(g * _G, (g + 1) * _G)
        w2s[sl, :] = (w2_ref[0, sl, :].astype(jnp.float32)
                      * s2_ref[0, g, :][None, :]).astype(jnp.bfloat16)

    dot = functools.partial(
        jax.lax.dot_general,
        dimension_numbers=(((1,), (0,)), ((), ())),
        preferred_element_type=jnp.float32,
    )

    aq = (jnp.clip(hs_ref[...] / p13, -448.0, 448.0)
          .astype(jnp.float8_e4m3fn).astype(jnp.bfloat16))
    fc1 = dot(aq, w1s[...]) * p13
    gate = dot(aq, w3s[...]) * p13
    h2 = fc1 * (gate * jax.nn.sigmoid(gate))

    aq2 = (jnp.clip(h2 / p2, -448.0, 448.0)
           .astype(jnp.float8_e4m3fn).astype(jnp.bfloat16))
    fc2 = dot(aq2, w2s[...]) * p2

    out_ref[...] += fc2 * fscale


def kernel(hidden_states, router_logits, W13, W2, S13, S2, P13, P2):
    hs = hidden_states.reshape(-1, _H)
    p13b = jnp.broadcast_to(P13[:, None, None], (_E, 8, 128))
    p2b = jnp.broadcast_to(P2[:, None, None], (_E, 8, 128))

    out = pl.pallas_call(
        _moe_body,
        grid=(_E,),
        in_specs=[
            pl.BlockSpec((_T, _H), lambda e: (0, 0)),
            pl.BlockSpec((_T, _E), lambda e: (0, 0)),
            pl.BlockSpec((1, _H, 2 * _I), lambda e: (e, 0, 0)),
            pl.BlockSpec((1, _NG, 2 * _I), lambda e: (e, 0, 0)),
            pl.BlockSpec((1, _I, _H), lambda e: (e, 0, 0)),
            pl.BlockSpec((1, _NG2, _H), lambda e: (e, 0, 0)),
            pl.BlockSpec((1, 8, 128), lambda e: (e, 0, 0)),
            pl.BlockSpec((1, 8, 128), lambda e: (e, 0, 0)),
        ],
        out_specs=pl.BlockSpec((_T, _H), lambda e: (0, 0)),
        out_shape=jax.ShapeDtypeStruct((_T, _H), jnp.float32),
        scratch_shapes=[
            pltpu.VMEM((_H, _I), jnp.bfloat16),
            pltpu.VMEM((_H, _I), jnp.bfloat16),
            pltpu.VMEM((_I, _H), jnp.bfloat16),
        ],
        compiler_params=pltpu.CompilerParams(
            dimension_semantics=("arbitrary",)),
    )(hs, router_logits, W13, S13, W2, S2, p13b, p2b)

    return out.reshape(hidden_states.shape)


# speedup vs baseline: 2.6960x; 1.0188x over previous
import functools

import jax
import jax.numpy as jnp
from jax.experimental import pallas as pl
from jax.experimental.pallas import tpu as pltpu

_E = 8
_H = 1024
_I = 2048
_G = 128
_T = 256

_NG = _H // _G
_NG2 = _I // _G


def _moe_body(hs_ref, logits_ref, w13_ref, s13_ref, w2_ref, s2_ref,
              p13_ref, p2_ref, out_ref, w1s, w3s, w2s):
    e = pl.program_id(0)

    @pl.when(e == 0)
    def _init():
        out_ref[...] = jnp.zeros_like(out_ref)

    logits = logits_ref[...]
    iota = jax.lax.broadcasted_iota(jnp.int32, logits.shape, 1)
    m1 = jnp.max(logits, axis=1, keepdims=True)
    a1 = jnp.min(jnp.where(logits == m1, iota, _E), axis=1, keepdims=True)
    masked = jnp.where(iota == a1, -jnp.inf, logits)
    m2 = jnp.max(masked, axis=1, keepdims=True)
    a2 = jnp.min(jnp.where(masked == m2, iota, _E), axis=1, keepdims=True)
    p_top = jax.nn.sigmoid(m1 - m2)
    p_snd = jax.nn.sigmoid(m2 - m1)
    fscale = (jnp.where(a1 == e, p_top, 0.0)
              + jnp.where(a2 == e, p_snd, 0.0))

    p13 = p13_ref[0, 0, 0]
    p2 = p2_ref[0, 0, 0]

    for g in range(_NG):
        sl = slice(g * _G, (g + 1) * _G)
        w1s[sl, :] = (w13_ref[0, sl, :_I].astype(jnp.float32)
                      * s13_ref[0, g, :_I][None, :])
        w3s[sl, :] = (w13_ref[0, sl, _I:].astype(jnp.float32)
                      * s13_ref[0, g, _I:][None, :])
    for g in range(_NG2):
        sl = slice(g * _G, (g + 1) * _G)
        w2s[sl, :] = (w2_ref[0, sl, :].astype(jnp.float32)
                      * s2_ref[0, g, :][None, :])

    dot = functools.partial(
        jax.lax.dot_general,
        dimension_numbers=(((1,), (0,)), ((), ())),
        preferred_element_type=jnp.float32,
    )

    aq = (jnp.clip(hs_ref[...] / p13, -448.0, 448.0)
          .astype(jnp.float8_e4m3fn).astype(jnp.float32))
    fc1 = dot(aq, w1s[...]) * p13
    gate = dot(aq, w3s[...]) * p13
    h2 = fc1 * (gate * jax.nn.sigmoid(gate))

    aq2 = (jnp.clip(h2 / p2, -448.0, 448.0)
           .astype(jnp.float8_e4m3fn).astype(jnp.float32))
    fc2 = dot(aq2, w2s[...]) * p2

    out_ref[...] += fc2 * fscale


def kernel(hidden_states, router_logits, W13, W2, S13, S2, P13, P2):
    hs = hidden_states.reshape(-1, _H)
    p13b = jnp.broadcast_to(P13[:, None, None], (_E, 8, 128))
    p2b = jnp.broadcast_to(P2[:, None, None], (_E, 8, 128))

    out = pl.pallas_call(
        _moe_body,
        grid=(_E,),
        in_specs=[
            pl.BlockSpec((_T, _H), lambda e: (0, 0)),
            pl.BlockSpec((_T, _E), lambda e: (0, 0)),
            pl.BlockSpec((1, _H, 2 * _I), lambda e: (e, 0, 0)),
            pl.BlockSpec((1, _NG, 2 * _I), lambda e: (e, 0, 0)),
            pl.BlockSpec((1, _I, _H), lambda e: (e, 0, 0)),
            pl.BlockSpec((1, _NG2, _H), lambda e: (e, 0, 0)),
            pl.BlockSpec((1, 8, 128), lambda e: (e, 0, 0)),
            pl.BlockSpec((1, 8, 128), lambda e: (e, 0, 0)),
        ],
        out_specs=pl.BlockSpec((_T, _H), lambda e: (0, 0)),
        out_shape=jax.ShapeDtypeStruct((_T, _H), jnp.float32),
        scratch_shapes=[
            pltpu.VMEM((_H, _I), jnp.float32),
            pltpu.VMEM((_H, _I), jnp.float32),
            pltpu.VMEM((_I, _H), jnp.float32),
        ],
        compiler_params=pltpu.CompilerParams(
            dimension_semantics=("arbitrary",)),
    )(hs, router_logits, W13, S13, W2, S2, p13b, p2b)

    return out.reshape(hidden_states.shape)
